# baseline (speedup 1.0000x reference)
import functools

import jax
import jax.numpy as jnp
from jax import lax
from jax.experimental import pallas as pl
from jax.experimental.pallas import tpu as pltpu
from jax.experimental.pallas import tpu_sc as plsc

_NBUF = 3


def _tc_fill_body(emb_ref, out_ref):
    out_ref[...] = emb_ref[...][None]


def _tc_combine_body(sc_ref, tcout_ref, out_ref):
    del tcout_ref
    out_ref[...] = sc_ref[...][None]


def kernel(input_ids, embeddings):
    B, S = input_ids.shape
    D = embeddings.shape[1]
    NC, NS = 2, 16
    NW = NC * NS
    rows_per_w = S // NW
    CHUNK = 32
    n_chunks = rows_per_w // CHUNK

    mesh = plsc.VectorSubcoreMesh(core_axis_name="c", subcore_axis_name="s")

    @functools.partial(
        pl.kernel,
        mesh=mesh,
        out_type=jax.ShapeDtypeStruct((S, D), embeddings.dtype),
        scratch_types=(
            [pltpu.VMEM((CHUNK, D), jnp.float32) for _ in range(_NBUF)]
            + [pltpu.SemaphoreType.DMA for _ in range(2 * _NBUF)]
        ),
    )
    def sc_slice(emb_hbm, out_hbm, *scratch):
        bufs = scratch[:_NBUF]
        rsems = scratch[_NBUF:2 * _NBUF]
        wsems = scratch[2 * _NBUF:]
        wid = lax.axis_index("s") * NC + lax.axis_index("c")
        base = wid * rows_per_w

        rcopies = [None] * n_chunks
        wcopies = [None] * n_chunks
        for c in range(min(_NBUF, n_chunks)):
            rcopies[c] = pltpu.async_copy(
                emb_hbm.at[pl.ds(base + c * CHUNK, CHUNK)], bufs[c], rsems[c])
        for c in range(n_chunks):
            i = c % _NBUF
            r0 = base + c * CHUNK
            rcopies[c].wait()
            wcopies[c] = pltpu.async_copy(
                bufs[i], out_hbm.at[pl.ds(r0, CHUNK)], wsems[i])
            nxt = c + _NBUF
            if nxt < n_chunks:
                wcopies[c].wait()
                rcopies[nxt] = pltpu.async_copy(
                    emb_hbm.at[pl.ds(base + nxt * CHUNK, CHUNK)], bufs[i], rsems[i])
        for c in range(n_chunks):
            if wcopies[c] is not None and c + _NBUF >= n_chunks:
                wcopies[c].wait()

    sc_out = sc_slice(embeddings)

    BS = 2048
    tc_out = pl.pallas_call(
        _tc_fill_body,
        grid=(S // BS, B - 1),
        in_specs=[pl.BlockSpec((BS, D), lambda i, b: (i, 0))],
        out_specs=pl.BlockSpec((1, BS, D), lambda i, b: (b, i, 0)),
        out_shape=jax.ShapeDtypeStruct((B, S, D), embeddings.dtype),
    )(embeddings)

    out = pl.pallas_call(
        _tc_combine_body,
        grid=(S // BS,),
        in_specs=[
            pl.BlockSpec((BS, D), lambda i: (i, 0)),
            pl.BlockSpec(memory_space=pl.ANY),
        ],
        out_specs=pl.BlockSpec((1, BS, D), lambda i: (B - 1, i, 0)),
        out_shape=jax.ShapeDtypeStruct((B, S, D), embeddings.dtype),
        input_output_aliases={1: 0},
    )(sc_out, tc_out)
    return out


# speedup vs baseline: 1.2837x; 1.2837x over previous
import functools

import jax
import jax.numpy as jnp
from jax import lax
from jax.experimental import pallas as pl
from jax.experimental.pallas import tpu as pltpu
from jax.experimental.pallas import tpu_sc as plsc

_NBUF = 3


def kernel(input_ids, embeddings):
    B, S = input_ids.shape
    D = embeddings.shape[1]
    NC, NS = 2, 16
    NW = NC * NS
    rows_per_w = S // NW
    CHUNK = 32
    n_chunks = rows_per_w // CHUNK

    mesh = plsc.VectorSubcoreMesh(core_axis_name="c", subcore_axis_name="s")

    @functools.partial(
        pl.kernel,
        mesh=mesh,
        out_type=jax.ShapeDtypeStruct((B, S, D), embeddings.dtype),
        scratch_types=(
            [pltpu.VMEM((CHUNK, D), jnp.float32) for _ in range(_NBUF)]
            + [pltpu.SemaphoreType.DMA for _ in range(2 * _NBUF)]
        ),
    )
    def sc_copy(emb_hbm, out_hbm, *scratch):
        bufs = scratch[:_NBUF]
        rsems = scratch[_NBUF:2 * _NBUF]
        wsems = scratch[2 * _NBUF:]
        wid = lax.axis_index("s") * NC + lax.axis_index("c")
        base = wid * rows_per_w

        rcopies = [None] * n_chunks
        wcopies = [[] for _ in range(_NBUF)]
        for c in range(min(_NBUF, n_chunks)):
            rcopies[c] = pltpu.async_copy(
                emb_hbm.at[pl.ds(base + c * CHUNK, CHUNK)], bufs[c], rsems[c])
        for c in range(n_chunks):
            i = c % _NBUF
            r0 = base + c * CHUNK
            rcopies[c].wait()
            for b in range(B):
                wcopies[i].append(pltpu.async_copy(
                    bufs[i], out_hbm.at[b, pl.ds(r0, CHUNK)], wsems[i]))
            nxt = c + _NBUF
            if nxt < n_chunks:
                for wc in wcopies[i]:
                    wc.wait()
                wcopies[i] = []
                rcopies[nxt] = pltpu.async_copy(
                    emb_hbm.at[pl.ds(base + nxt * CHUNK, CHUNK)], bufs[i], rsems[i])
        for i in range(_NBUF):
            for wc in wcopies[i]:
                wc.wait()

    return sc_copy(embeddings)
